# baseline (speedup 1.0000x reference)
import jax
import jax.numpy as jnp
from jax.experimental import pallas as pl
from jax.experimental.pallas import tpu as pltpu

N = 2048
K = 5
BR = 512
NI = N // BR
F = 2
NH = N // F
_PADR = 8


def _knn_std_kernel(yh_ref, out_ref, gacc_ref, x2acc_ref, y2_ref, acc_ref):
    f = pl.program_id(0)
    i = pl.program_id(1)

    yh = yh_ref[...]
    xh = yh_ref[pl.ds(pl.multiple_of(i * BR, BR), BR), :]

    gp = jax.lax.dot_general(
        yh, xh, (((1,), (1,)), ((), ())), preferred_element_type=jnp.float32
    )

    xsq = xh * xh
    ones = jnp.ones((8, NH), jnp.float32)
    x2p = jax.lax.dot_general(
        ones, xsq, (((1,), (1,)), ((), ())),
        preferred_element_type=jnp.float32,
    )

    @pl.when(i == 0)
    def _():
        part = jnp.sum(yh * yh, axis=1, keepdims=True)

        @pl.when(f == 0)
        def _():
            y2_ref[...] = part

        @pl.when(f == 1)
        def _():
            y2_ref[...] = y2_ref[...] + part

    @pl.when(f == 0)
    def _():
        gacc_ref[i] = gp
        x2acc_ref[i] = x2p

    @pl.when(f == 1)
    def _():
        g = gacc_ref[i] + gp
        st = y2_ref[...] - 2.0 * g

        inf = jnp.full((_PADR, BR), jnp.inf, jnp.float32)
        s = [inf] * K
        for r in range(N // _PADR):
            v = st[r * _PADR:(r + 1) * _PADR, :]
            for t in range(K - 1, -1, -1):
                lo = jnp.minimum(s[t], v)
                v = jnp.maximum(s[t], v)
                s[t] = lo

        cand = jnp.concatenate(s, axis=0)
        H = K * _PADR
        iota = jax.lax.broadcasted_iota(jnp.int32, (H, BR), 0)
        row = jax.lax.broadcasted_iota(jnp.int32, (_PADR, BR), 0)
        sel = jnp.full((_PADR, BR), jnp.inf, jnp.float32)
        for t in range(K):
            m = jnp.min(cand, axis=0, keepdims=True)
            r0 = jnp.min(jnp.where(cand == m, iota, H), axis=0,
                         keepdims=True)
            cand = jnp.where(iota == r0, jnp.inf, cand)
            sel = jnp.where(row == t, m, sel)

        x2 = (x2acc_ref[i] + x2p)[0:1, :]
        d2 = jnp.maximum(x2 + sel, 0.0)
        knn = jnp.where(d2 > 0.0,
                        jnp.sqrt(jnp.where(d2 > 0.0, d2, 1.0)), 0.0)
        valid = row < K
        knn = jnp.where(valid, knn, 0.0)
        nb = jnp.float32(BR * K)
        mean_b = jnp.sum(knn) / nb
        dev = jnp.where(valid, knn - mean_b, 0.0)
        m2_b = jnp.sum(dev * dev)

        @pl.when(i == 0)
        def _():
            acc_ref[0] = nb
            acc_ref[1] = mean_b
            acc_ref[2] = m2_b

        @pl.when(i > 0)
        def _():
            na = acc_ref[0]
            mean_a = acc_ref[1]
            m2_a = acc_ref[2]
            n = na + nb
            delta = mean_b - mean_a
            acc_ref[0] = n
            acc_ref[1] = mean_a + delta * (nb / n)
            acc_ref[2] = m2_a + m2_b + delta * delta * (na * nb / n)

        @pl.when(i == NI - 1)
        def _():
            n = acc_ref[0]
            out_ref[...] = jnp.full(
                (1, 1), -jnp.sqrt(acc_ref[2] / (n - 1.0)), jnp.float32
            )


def kernel(latent):
    sample = latent[0]
    out = pl.pallas_call(
        _knn_std_kernel,
        grid=(F, NI),
        in_specs=[
            pl.BlockSpec((N, NH), lambda f, i: (0, f)),
        ],
        out_specs=pl.BlockSpec((1, 1), lambda f, i: (0, 0)),
        out_shape=jax.ShapeDtypeStruct((1, 1), jnp.float32),
        scratch_shapes=[
            pltpu.VMEM((NI, N, BR), jnp.float32),
            pltpu.VMEM((NI, 8, BR), jnp.float32),
            pltpu.VMEM((N, 1), jnp.float32),
            pltpu.SMEM((4,), jnp.float32),
        ],
    )(sample)
    return out[0, 0]


# speedup vs baseline: 1.0603x; 1.0603x over previous
import jax
import jax.numpy as jnp
from jax.experimental import pallas as pl
from jax.experimental.pallas import tpu as pltpu

N = 2048
K = 5
BR = 1024
NI = N // BR
_PADR = 8


def _knn_std_kernel(y_ref, out_ref, y2_ref, acc_ref):
    i = pl.program_id(0)

    y = y_ref[...]
    x = y_ref[pl.ds(pl.multiple_of(i * BR, BR), BR), :]

    g = jax.lax.dot_general(
        y, x, (((1,), (1,)), ((), ())), preferred_element_type=jnp.float32
    )

    @pl.when(i == 0)
    def _():
        y2_ref[...] = jnp.sum(y * y, axis=1, keepdims=True)

    y2 = y2_ref[...]
    st = y2 - 2.0 * g

    inf = jnp.full((_PADR, BR), jnp.inf, jnp.float32)
    s = [inf] * K
    for r in range(N // _PADR):
        v = st[r * _PADR:(r + 1) * _PADR, :]
        for t in range(K - 1, -1, -1):
            lo = jnp.minimum(s[t], v)
            v = jnp.maximum(s[t], v)
            s[t] = lo

    cand = jnp.concatenate(s, axis=0)
    H = K * _PADR
    iota = jax.lax.broadcasted_iota(jnp.int32, (H, BR), 0)
    row = jax.lax.broadcasted_iota(jnp.int32, (_PADR, BR), 0)
    sel = jnp.full((_PADR, BR), jnp.inf, jnp.float32)
    for t in range(K):
        m = jnp.min(cand, axis=0, keepdims=True)
        r0 = jnp.min(jnp.where(cand == m, iota, H), axis=0, keepdims=True)
        cand = jnp.where(iota == r0, jnp.inf, cand)
        sel = jnp.where(row == t, m, sel)

    ones = jnp.ones((8, N), jnp.float32)
    x2 = jax.lax.dot_general(
        ones, x * x, (((1,), (1,)), ((), ())),
        preferred_element_type=jnp.float32,
    )[0:1, :]
    d2 = jnp.maximum(x2 + sel, 0.0)
    knn = jnp.where(d2 > 0.0, jnp.sqrt(jnp.where(d2 > 0.0, d2, 1.0)), 0.0)
    valid = row < K
    knn = jnp.where(valid, knn, 0.0)
    nb = jnp.float32(BR * K)
    mean_b = jnp.sum(knn) / nb
    dev = jnp.where(valid, knn - mean_b, 0.0)
    m2_b = jnp.sum(dev * dev)

    @pl.when(i == 0)
    def _():
        acc_ref[0] = nb
        acc_ref[1] = mean_b
        acc_ref[2] = m2_b

    @pl.when(i > 0)
    def _():
        na = acc_ref[0]
        mean_a = acc_ref[1]
        m2_a = acc_ref[2]
        n = na + nb
        delta = mean_b - mean_a
        acc_ref[0] = n
        acc_ref[1] = mean_a + delta * (nb / n)
        acc_ref[2] = m2_a + m2_b + delta * delta * (na * nb / n)

    @pl.when(i == NI - 1)
    def _():
        n = acc_ref[0]
        out_ref[...] = jnp.full(
            (1, 1), -jnp.sqrt(acc_ref[2] / (n - 1.0)), jnp.float32
        )


def kernel(latent):
    sample = latent[0]
    out = pl.pallas_call(
        _knn_std_kernel,
        grid=(NI,),
        in_specs=[
            pl.BlockSpec((N, N), lambda i: (0, 0)),
        ],
        out_specs=pl.BlockSpec((1, 1), lambda i: (0, 0)),
        out_shape=jax.ShapeDtypeStruct((1, 1), jnp.float32),
        scratch_shapes=[
            pltpu.VMEM((N, 1), jnp.float32),
            pltpu.SMEM((4,), jnp.float32),
        ],
    )(sample)
    return out[0, 0]
